# baseline (speedup 1.0000x reference)
import functools
import jax
import jax.numpy as jnp
from jax import lax
from jax.experimental import pallas as pl
from jax.experimental.pallas import tpu as pltpu
from jax.experimental.pallas import tpu_sc as plsc

_N = 10000
_K = 6
_KG = _K - 1
_NPAD = 10240
_E = _NPAD * _KG
_NW = 32
_EPW = _E // _NW
_QB = 1000
_CB = 2048


def _knn_kernel(q_ref, post_ref, w1_ref, b1_ref, wp_ref,
                nbr_ref, y1_ref, p1_ref):
    q = q_ref[...]
    d = None
    for c in range(3):
        diff = q[:, c:c + 1] - post_ref[c:c + 1, :]
        sq = diff * diff
        d = sq if d is None else d + sq
    cols = lax.broadcasted_iota(jnp.int32, d.shape, 1)
    rows = lax.broadcasted_iota(jnp.int32, d.shape, 0) + pl.program_id(0) * _QB
    d = jnp.where(cols == rows, jnp.float32(jnp.inf), d)
    idx_cols = []
    for _ in range(_KG):
        m = jnp.min(d, axis=1, keepdims=True)
        sel = jnp.where(d == m, cols, jnp.int32(2**30))
        idx = jnp.min(sel, axis=1, keepdims=True)
        idx_cols.append(idx)
        d = jnp.where(cols == idx, jnp.float32(jnp.inf), d)
    nbr_ref[...] = jnp.concatenate(idx_cols, axis=1)
    p1 = jnp.dot(q, wp_ref[...], preferred_element_type=jnp.float32)
    y1_ref[...] = jnp.dot(q, w1_ref[...],
                          preferred_element_type=jnp.float32) + b1_ref[...]
    p1_ref[...] = p1


def _knn(pos, w1sum, b1, w1p):
    post = pos.T
    grid = _N // _QB
    return pl.pallas_call(
        _knn_kernel,
        grid=(grid,),
        in_specs=[
            pl.BlockSpec((_QB, 3), lambda i: (i, 0)),
            pl.BlockSpec((3, _N), lambda i: (0, 0)),
            pl.BlockSpec((3, 32), lambda i: (0, 0)),
            pl.BlockSpec((1, 32), lambda i: (0, 0)),
            pl.BlockSpec((3, 32), lambda i: (0, 0)),
        ],
        out_specs=[
            pl.BlockSpec((_QB, _KG), lambda i: (i, 0)),
            pl.BlockSpec((_QB, 32), lambda i: (i, 0)),
            pl.BlockSpec((_QB, 32), lambda i: (i, 0)),
        ],
        out_shape=[
            jax.ShapeDtypeStruct((_N, _KG), jnp.int32),
            jax.ShapeDtypeStruct((_N, 32), jnp.float32),
            jax.ShapeDtypeStruct((_N, 32), jnp.float32),
        ],
        compiler_params=pltpu.CompilerParams(
            dimension_semantics=("arbitrary",)),
    )(pos, post, w1sum, b1, w1p)


@functools.lru_cache(maxsize=None)
def _make_sc_gather(d_feat, n_chunks):
    rpc = _EPW // n_chunks
    mesh = plsc.VectorSubcoreMesh(core_axis_name="c", subcore_axis_name="s")

    @functools.partial(
        pl.kernel,
        out_type=jax.ShapeDtypeStruct((_E, d_feat), jnp.float32),
        mesh=mesh,
        scratch_types=[
            pltpu.VMEM((_EPW,), jnp.int32),
            pltpu.VMEM((rpc, d_feat), jnp.float32),
            pltpu.VMEM((rpc, d_feat), jnp.float32),
            pltpu.SemaphoreType.DMA,
            pltpu.SemaphoreType.DMA,
            pltpu.SemaphoreType.DMA,
            pltpu.SemaphoreType.DMA,
        ],
        compiler_params=pltpu.CompilerParams(use_tc_tiling_on_sc=False),
    )
    def gather(table_hbm, idx_hbm, out_hbm, idx_v, buf0, buf1,
               gs0, gs1, ws0, ws1):
        wid = lax.axis_index("s") * 2 + lax.axis_index("c")
        base = wid * _EPW
        bufs = (buf0, buf1)
        gsems = (gs0, gs1)
        wsems = (ws0, ws1)
        pltpu.sync_copy(idx_hbm.at[pl.ds(base, _EPW)], idx_v)
        ghandles = [None, None]
        whandles = [None, None]
        for c in range(n_chunks):
            b = c & 1
            if whandles[b] is not None:
                whandles[b].wait()
            ghandles[b] = pltpu.async_copy(
                table_hbm.at[idx_v.at[pl.ds(c * rpc, rpc)]], bufs[b], gsems[b])
            if c >= 1:
                pb = (c - 1) & 1
                ghandles[pb].wait()
                whandles[pb] = pltpu.async_copy(
                    bufs[pb], out_hbm.at[pl.ds(base + (c - 1) * rpc, rpc)],
                    wsems[pb])
        lb = (n_chunks - 1) & 1
        ghandles[lb].wait()
        whandles[lb] = pltpu.async_copy(
            bufs[lb], out_hbm.at[pl.ds(base + (n_chunks - 1) * rpc, rpc)],
            wsems[lb])
        for w in whandles:
            if w is not None:
                w.wait()

    return gather


def _gather32(table, idx):
    return _make_sc_gather(32, 2)(table, idx)


def _gather64(table, idx):
    return _make_sc_gather(64, 2)(table, idx)


def _gather128(table, idx):
    return _make_sc_gather(128, 4)(table, idx)


def _conv_mid_kernel(g_ref, y_ref, p_ref, pos_ref, wb_ref, bb_ref,
                     wnx_ref, wnp_ref, bn_ref,
                     h_ref, yn_ref, pn_ref):
    pb = p_ref[...]
    wb = wb_ref[...]
    e0 = jnp.maximum(y_ref[...] - pb, 0.0)
    acc = jnp.dot(e0, wb, preferred_element_type=jnp.float32)
    for j in range(_KG):
        e = jnp.maximum(g_ref[:, j, :] - pb, 0.0)
        t = jnp.dot(e, wb, preferred_element_type=jnp.float32)
        acc = jnp.maximum(acc, t)
    h = jnp.maximum(acc + bb_ref[...], 0.0)
    h_ref[...] = h
    pn = jnp.dot(pos_ref[...], wnp_ref[...], preferred_element_type=jnp.float32)
    yn_ref[...] = jnp.dot(h, wnx_ref[...],
                          preferred_element_type=jnp.float32) + pn + bn_ref[...]
    pn_ref[...] = pn


def _conv_last_kernel(g_ref, y_ref, p_ref, wb_ref, bb_ref, h_ref):
    pb = p_ref[...]
    wb = wb_ref[...]
    e0 = jnp.maximum(y_ref[...] - pb, 0.0)
    acc = jnp.dot(e0, wb, preferred_element_type=jnp.float32)
    for j in range(_KG):
        e = jnp.maximum(g_ref[:, j, :] - pb, 0.0)
        t = jnp.dot(e, wb, preferred_element_type=jnp.float32)
        acc = jnp.maximum(acc, t)
    h_ref[...] = jnp.maximum(acc + bb_ref[...], 0.0)


def _conv_mid(g, y, p, pos_pad, wb, bb, wnx, wnp, bn):
    fi, fo = wb.shape
    fn = wnx.shape[1]
    grid = _NPAD // _CB
    full = lambda i: (0, 0)
    return pl.pallas_call(
        _conv_mid_kernel,
        grid=(grid,),
        in_specs=[
            pl.BlockSpec((_CB, _KG, fi), lambda i: (i, 0, 0)),
            pl.BlockSpec((_CB, fi), lambda i: (i, 0)),
            pl.BlockSpec((_CB, fi), lambda i: (i, 0)),
            pl.BlockSpec((_CB, 3), lambda i: (i, 0)),
            pl.BlockSpec(wb.shape, full),
            pl.BlockSpec((1, fo), full),
            pl.BlockSpec(wnx.shape, full),
            pl.BlockSpec((3, fn), full),
            pl.BlockSpec((1, fn), full),
        ],
        out_specs=[
            pl.BlockSpec((_CB, fo), lambda i: (i, 0)),
            pl.BlockSpec((_CB, fn), lambda i: (i, 0)),
            pl.BlockSpec((_CB, fn), lambda i: (i, 0)),
        ],
        out_shape=[
            jax.ShapeDtypeStruct((_NPAD, fo), jnp.float32),
            jax.ShapeDtypeStruct((_NPAD, fn), jnp.float32),
            jax.ShapeDtypeStruct((_NPAD, fn), jnp.float32),
        ],
        compiler_params=pltpu.CompilerParams(
            dimension_semantics=("arbitrary",)),
    )(g, y, p, pos_pad, wb, bb, wnx, wnp, bn)


def _conv_last(g, y, p, wb, bb):
    fi, fo = wb.shape
    grid = _NPAD // _CB
    full = lambda i: (0, 0)
    return pl.pallas_call(
        _conv_last_kernel,
        grid=(grid,),
        in_specs=[
            pl.BlockSpec((_CB, _KG, fi), lambda i: (i, 0, 0)),
            pl.BlockSpec((_CB, fi), lambda i: (i, 0)),
            pl.BlockSpec((_CB, fi), lambda i: (i, 0)),
            pl.BlockSpec(wb.shape, full),
            pl.BlockSpec((1, fo), full),
        ],
        out_specs=pl.BlockSpec((_CB, fo), lambda i: (i, 0)),
        out_shape=jax.ShapeDtypeStruct((_NPAD, fo), jnp.float32),
        compiler_params=pltpu.CompilerParams(
            dimension_semantics=("arbitrary",)),
    )(g, y, p, wb, bb)


def kernel(pos, w1a, b1a, w1b, b1b, w2a, b2a, w2b, b2b, w3a, b3a, w3b, b3b):
    f = jnp.float32
    w1x, w1p = w1a[:3], w1a[3:]
    w2x, w2p = w2a[:32], w2a[32:]
    w3x, w3p = w3a[:64], w3a[64:]

    nbr, y1, p1 = _knn(pos.astype(f), (w1x + w1p).astype(f),
                       b1a.reshape(1, -1).astype(f), w1p.astype(f))

    src = jnp.concatenate(
        [nbr.reshape(-1), jnp.zeros((_E - _N * _KG,), jnp.int32)])
    zpad = lambda a: jnp.pad(a, ((0, _NPAD - _N), (0, 0)))
    pos_pad = zpad(pos.astype(f))

    g1 = _gather32(y1, src).reshape(_NPAD, _KG, 32)
    h1, y2, p2 = _conv_mid(g1, zpad(y1), zpad(p1), pos_pad,
                           w1b, b1b.reshape(1, -1),
                           w2x, w2p, b2a.reshape(1, -1))

    g2 = _gather64(y2, src).reshape(_NPAD, _KG, 64)
    h2, y3, p3 = _conv_mid(g2, y2, p2, pos_pad, w2b, b2b.reshape(1, -1),
                           w3x, w3p, b3a.reshape(1, -1))

    g3 = _gather128(y3, src).reshape(_NPAD, _KG, 128)
    h3 = _conv_last(g3, y3, p3, w3b, b3b.reshape(1, -1))
    return h3[:_N]


# speedup vs baseline: 1.1576x; 1.1576x over previous
import functools
import jax
import jax.numpy as jnp
from jax import lax
from jax.experimental import pallas as pl
from jax.experimental.pallas import tpu as pltpu
from jax.experimental.pallas import tpu_sc as plsc

_N = 10000
_K = 6
_KG = _K - 1
_NPAD = 10240
_E = _NPAD * _KG
_NW = 32
_EPW = _E // _NW
_QB = 400
_CB = 2048


def _knn_kernel(q_ref, post_ref, w1_ref, b1_ref, wp_ref,
                nbr_ref, y1_ref, p1_ref):
    q = q_ref[...]
    d = None
    for c in range(3):
        diff = q[:, c:c + 1] - post_ref[c:c + 1, :]
        sq = diff * diff
        d = sq if d is None else d + sq
    cols = lax.broadcasted_iota(jnp.int32, d.shape, 1)
    rows = lax.broadcasted_iota(jnp.int32, d.shape, 0) + pl.program_id(0) * _QB
    d = jnp.where(cols == rows, jnp.float32(jnp.inf), d)
    idx_cols = []
    for _ in range(_KG):
        m = jnp.min(d, axis=1, keepdims=True)
        sel = jnp.where(d == m, cols, jnp.int32(2**30))
        idx = jnp.min(sel, axis=1, keepdims=True)
        idx_cols.append(idx)
        d = jnp.where(cols == idx, jnp.float32(jnp.inf), d)
    nbr_ref[...] = jnp.concatenate(idx_cols, axis=1)
    p1 = jnp.dot(q, wp_ref[...], preferred_element_type=jnp.float32)
    y1_ref[...] = jnp.dot(q, w1_ref[...],
                          preferred_element_type=jnp.float32) + b1_ref[...]
    p1_ref[...] = p1


def _knn(pos, w1sum, b1, w1p):
    post = pos.T
    grid = _N // _QB
    return pl.pallas_call(
        _knn_kernel,
        grid=(grid,),
        in_specs=[
            pl.BlockSpec((_QB, 3), lambda i: (i, 0)),
            pl.BlockSpec((3, _N), lambda i: (0, 0)),
            pl.BlockSpec((3, 32), lambda i: (0, 0)),
            pl.BlockSpec((1, 32), lambda i: (0, 0)),
            pl.BlockSpec((3, 32), lambda i: (0, 0)),
        ],
        out_specs=[
            pl.BlockSpec((_QB, _KG), lambda i: (i, 0)),
            pl.BlockSpec((_QB, 32), lambda i: (i, 0)),
            pl.BlockSpec((_QB, 32), lambda i: (i, 0)),
        ],
        out_shape=[
            jax.ShapeDtypeStruct((_N, _KG), jnp.int32),
            jax.ShapeDtypeStruct((_N, 32), jnp.float32),
            jax.ShapeDtypeStruct((_N, 32), jnp.float32),
        ],
        compiler_params=pltpu.CompilerParams(
            dimension_semantics=("arbitrary",)),
    )(pos, post, w1sum, b1, w1p)


@functools.lru_cache(maxsize=None)
def _make_sc_gather(d_feat, n_chunks):
    rpc = _EPW // n_chunks
    mesh = plsc.VectorSubcoreMesh(core_axis_name="c", subcore_axis_name="s")

    @functools.partial(
        pl.kernel,
        out_type=jax.ShapeDtypeStruct((_E, d_feat), jnp.float32),
        mesh=mesh,
        scratch_types=[
            pltpu.VMEM((_EPW,), jnp.int32),
            pltpu.VMEM((rpc, d_feat), jnp.float32),
            pltpu.VMEM((rpc, d_feat), jnp.float32),
            pltpu.SemaphoreType.DMA,
            pltpu.SemaphoreType.DMA,
            pltpu.SemaphoreType.DMA,
            pltpu.SemaphoreType.DMA,
        ],
        compiler_params=pltpu.CompilerParams(use_tc_tiling_on_sc=False),
    )
    def gather(table_hbm, idx_hbm, out_hbm, idx_v, buf0, buf1,
               gs0, gs1, ws0, ws1):
        wid = lax.axis_index("s") * 2 + lax.axis_index("c")
        base = wid * _EPW
        bufs = (buf0, buf1)
        gsems = (gs0, gs1)
        wsems = (ws0, ws1)
        pltpu.sync_copy(idx_hbm.at[pl.ds(base, _EPW)], idx_v)
        ghandles = [None, None]
        whandles = [None, None]
        for c in range(n_chunks):
            b = c & 1
            if whandles[b] is not None:
                whandles[b].wait()
            ghandles[b] = pltpu.async_copy(
                table_hbm.at[idx_v.at[pl.ds(c * rpc, rpc)]], bufs[b], gsems[b])
            if c >= 1:
                pb = (c - 1) & 1
                ghandles[pb].wait()
                whandles[pb] = pltpu.async_copy(
                    bufs[pb], out_hbm.at[pl.ds(base + (c - 1) * rpc, rpc)],
                    wsems[pb])
        lb = (n_chunks - 1) & 1
        ghandles[lb].wait()
        whandles[lb] = pltpu.async_copy(
            bufs[lb], out_hbm.at[pl.ds(base + (n_chunks - 1) * rpc, rpc)],
            wsems[lb])
        for w in whandles:
            if w is not None:
                w.wait()

    return gather


@functools.lru_cache(maxsize=None)
def _make_sc_gather_half(d_feat, n_chunks, half):
    eh = _E // 2
    epw = eh // _NW
    rpc = epw // n_chunks
    mesh = plsc.VectorSubcoreMesh(core_axis_name="c", subcore_axis_name="s")

    @functools.partial(
        pl.kernel,
        out_type=jax.ShapeDtypeStruct((eh, d_feat), jnp.float32),
        mesh=mesh,
        scratch_types=[
            pltpu.VMEM((epw,), jnp.int32),
            pltpu.VMEM((rpc, d_feat), jnp.float32),
            pltpu.VMEM((rpc, d_feat), jnp.float32),
            pltpu.SemaphoreType.DMA,
            pltpu.SemaphoreType.DMA,
            pltpu.SemaphoreType.DMA,
            pltpu.SemaphoreType.DMA,
        ],
        compiler_params=pltpu.CompilerParams(use_tc_tiling_on_sc=False),
    )
    def gather(table_hbm, idx_hbm, out_hbm, idx_v, buf0, buf1,
               gs0, gs1, ws0, ws1):
        wid = lax.axis_index("s") * 2 + lax.axis_index("c")
        gbase = half * eh + wid * epw
        obase = wid * epw
        bufs = (buf0, buf1)
        gsems = (gs0, gs1)
        wsems = (ws0, ws1)
        pltpu.sync_copy(idx_hbm.at[pl.ds(gbase, epw)], idx_v)
        ghandles = [None, None]
        whandles = [None, None]
        for c in range(n_chunks):
            b = c & 1
            if whandles[b] is not None:
                whandles[b].wait()
            ghandles[b] = pltpu.async_copy(
                table_hbm.at[idx_v.at[pl.ds(c * rpc, rpc)]], bufs[b], gsems[b])
            if c >= 1:
                pb = (c - 1) & 1
                ghandles[pb].wait()
                whandles[pb] = pltpu.async_copy(
                    bufs[pb], out_hbm.at[pl.ds(obase + (c - 1) * rpc, rpc)],
                    wsems[pb])
        lb = (n_chunks - 1) & 1
        ghandles[lb].wait()
        whandles[lb] = pltpu.async_copy(
            bufs[lb], out_hbm.at[pl.ds(obase + (n_chunks - 1) * rpc, rpc)],
            wsems[lb])
        for w in whandles:
            if w is not None:
                w.wait()

    return gather


def _gather32(table, idx):
    return _make_sc_gather(32, 2)(table, idx)


def _gather64(table, idx):
    return _make_sc_gather(64, 2)(table, idx)


def _gather128(table, idx):
    return _make_sc_gather(128, 4)(table, idx)


def _conv_mid_kernel(g_ref, y_ref, p_ref, pos_ref, wb_ref, bb_ref,
                     wnx_ref, wnp_ref, bn_ref,
                     h_ref, yn_ref, pn_ref):
    pb = p_ref[...]
    wb = wb_ref[...]
    e0 = jnp.maximum(y_ref[...] - pb, 0.0)
    acc = jnp.dot(e0, wb, preferred_element_type=jnp.float32)
    for j in range(_KG):
        e = jnp.maximum(g_ref[:, j, :] - pb, 0.0)
        t = jnp.dot(e, wb, preferred_element_type=jnp.float32)
        acc = jnp.maximum(acc, t)
    h = jnp.maximum(acc + bb_ref[...], 0.0)
    h_ref[...] = h
    pn = jnp.dot(pos_ref[...], wnp_ref[...], preferred_element_type=jnp.float32)
    yn_ref[...] = jnp.dot(h, wnx_ref[...],
                          preferred_element_type=jnp.float32) + pn + bn_ref[...]
    pn_ref[...] = pn


def _conv_last_kernel(g_ref, y_ref, p_ref, wb_ref, bb_ref, h_ref):
    pb = p_ref[...]
    wb = wb_ref[...]
    e0 = jnp.maximum(y_ref[...] - pb, 0.0)
    acc = jnp.dot(e0, wb, preferred_element_type=jnp.float32)
    for j in range(_KG):
        e = jnp.maximum(g_ref[:, j, :] - pb, 0.0)
        t = jnp.dot(e, wb, preferred_element_type=jnp.float32)
        acc = jnp.maximum(acc, t)
    h_ref[...] = jnp.maximum(acc + bb_ref[...], 0.0)


def _conv_mid(g, y, p, pos_pad, wb, bb, wnx, wnp, bn):
    fi, fo = wb.shape
    fn = wnx.shape[1]
    grid = _NPAD // _CB
    full = lambda i: (0, 0)
    return pl.pallas_call(
        _conv_mid_kernel,
        grid=(grid,),
        in_specs=[
            pl.BlockSpec((_CB, _KG, fi), lambda i: (i, 0, 0)),
            pl.BlockSpec((_CB, fi), lambda i: (i, 0)),
            pl.BlockSpec((_CB, fi), lambda i: (i, 0)),
            pl.BlockSpec((_CB, 3), lambda i: (i, 0)),
            pl.BlockSpec(wb.shape, full),
            pl.BlockSpec((1, fo), full),
            pl.BlockSpec(wnx.shape, full),
            pl.BlockSpec((3, fn), full),
            pl.BlockSpec((1, fn), full),
        ],
        out_specs=[
            pl.BlockSpec((_CB, fo), lambda i: (i, 0)),
            pl.BlockSpec((_CB, fn), lambda i: (i, 0)),
            pl.BlockSpec((_CB, fn), lambda i: (i, 0)),
        ],
        out_shape=[
            jax.ShapeDtypeStruct((_NPAD, fo), jnp.float32),
            jax.ShapeDtypeStruct((_NPAD, fn), jnp.float32),
            jax.ShapeDtypeStruct((_NPAD, fn), jnp.float32),
        ],
        compiler_params=pltpu.CompilerParams(
            dimension_semantics=("arbitrary",)),
    )(g, y, p, pos_pad, wb, bb, wnx, wnp, bn)


def _conv_last(g, y, p, wb, bb):
    fi, fo = wb.shape
    grid = _NPAD // _CB
    full = lambda i: (0, 0)
    return pl.pallas_call(
        _conv_last_kernel,
        grid=(grid,),
        in_specs=[
            pl.BlockSpec((_CB, _KG, fi), lambda i: (i, 0, 0)),
            pl.BlockSpec((_CB, fi), lambda i: (i, 0)),
            pl.BlockSpec((_CB, fi), lambda i: (i, 0)),
            pl.BlockSpec(wb.shape, full),
            pl.BlockSpec((1, fo), full),
        ],
        out_specs=pl.BlockSpec((_CB, fo), lambda i: (i, 0)),
        out_shape=jax.ShapeDtypeStruct((_NPAD, fo), jnp.float32),
        compiler_params=pltpu.CompilerParams(
            dimension_semantics=("arbitrary",)),
    )(g, y, p, wb, bb)


def _conv_last_half(g, y, p, wb, bb, half):
    fi, fo = wb.shape
    nh = _NPAD // 2
    grid = nh // _CB
    off = half * grid
    full = lambda i: (0, 0)
    return pl.pallas_call(
        _conv_last_kernel,
        grid=(grid,),
        in_specs=[
            pl.BlockSpec((_CB, _KG, fi), lambda i: (i, 0, 0)),
            pl.BlockSpec((_CB, fi), lambda i: (i + off, 0)),
            pl.BlockSpec((_CB, fi), lambda i: (i + off, 0)),
            pl.BlockSpec(wb.shape, full),
            pl.BlockSpec((1, fo), full),
        ],
        out_specs=pl.BlockSpec((_CB, fo), lambda i: (i, 0)),
        out_shape=jax.ShapeDtypeStruct((nh, fo), jnp.float32),
        compiler_params=pltpu.CompilerParams(
            dimension_semantics=("arbitrary",)),
    )(g, y, p, wb, bb)


def kernel(pos, w1a, b1a, w1b, b1b, w2a, b2a, w2b, b2b, w3a, b3a, w3b, b3b):
    f = jnp.float32
    w1x, w1p = w1a[:3], w1a[3:]
    w2x, w2p = w2a[:32], w2a[32:]
    w3x, w3p = w3a[:64], w3a[64:]

    nbr, y1, p1 = _knn(pos.astype(f), (w1x + w1p).astype(f),
                       b1a.reshape(1, -1).astype(f), w1p.astype(f))

    src = jnp.concatenate(
        [nbr.reshape(-1), jnp.zeros((_E - _N * _KG,), jnp.int32)])
    zpad = lambda a: jnp.pad(a, ((0, _NPAD - _N), (0, 0)))
    pos_pad = zpad(pos.astype(f))

    g1 = _gather32(y1, src).reshape(_NPAD, _KG, 32)
    h1, y2, p2 = _conv_mid(g1, zpad(y1), zpad(p1), pos_pad,
                           w1b, b1b.reshape(1, -1),
                           w2x, w2p, b2a.reshape(1, -1))

    g2 = _gather64(y2, src).reshape(_NPAD, _KG, 64)
    h2, y3, p3 = _conv_mid(g2, y2, p2, pos_pad, w2b, b2b.reshape(1, -1),
                           w3x, w3p, b3a.reshape(1, -1))

    nh = _NPAD // 2
    g3a = _make_sc_gather_half(128, 2, 0)(y3, src).reshape(nh, _KG, 128)
    g3b = _make_sc_gather_half(128, 2, 1)(y3, src).reshape(nh, _KG, 128)
    h3a = _conv_last_half(g3a, y3, p3, w3b, b3b.reshape(1, -1), 0)
    h3b = _conv_last_half(g3b, y3, p3, w3b, b3b.reshape(1, -1), 1)
    return jnp.concatenate([h3a, h3b])[:_N]
